# baseline (speedup 1.0000x reference)
import functools

import jax
import jax.numpy as jnp
from jax import lax
from jax.experimental import pallas as pl
from jax.experimental.pallas import tpu as pltpu
from jax.experimental.pallas import tpu_sc as plsc

_N = 10000
_D = 128
_NC = 2
_NS = 16
_CHUNK = 128
_RPT = 8 * (-(-_N // (_NS * 8)))
_N_ACC = _NS * _RPT
_BR = 1000


@functools.cache
def _make_sc_seg(tpc):
    mesh = plsc.VectorSubcoreMesh(core_axis_name="c", subcore_axis_name="s")

    @functools.partial(
        pl.kernel,
        out_type=jax.ShapeDtypeStruct((2 * _N, _D), jnp.float32),
        mesh=mesh,
        scratch_types=[
            [pltpu.VMEM((2, _CHUNK), jnp.int32)] * 4,
            [pltpu.VMEM((_CHUNK, _D), jnp.float32)] * 2,
            pltpu.VMEM_SHARED((_N_ACC, _D), jnp.float32),
            [pltpu.SemaphoreType.DMA] * 4,
            [pltpu.SemaphoreType.DMA] * 2,
            [pltpu.SemaphoreType.DMA] * 2,
        ],
    )
    def seg(sup, edges, out, idx, rows, acc, isem, gsem, ssem):
        c = lax.axis_index("c")
        s = lax.axis_index("s")
        w = c * _NS + s

        zero16 = jnp.zeros((16,), jnp.float32)

        def _zrow(r, carry):
            for k in range(_D // 16):
                rows[0][r, pl.ds(16 * k, 16)] = zero16
            return carry

        lax.fori_loop(0, _CHUNK, _zrow, 0)
        zbase = s * _RPT
        nfull = _RPT // _CHUNK
        for m in range(nfull):
            pltpu.sync_copy(rows[0], acc.at[pl.ds(zbase + m * _CHUNK, _CHUNK)])
        rem = _RPT % _CHUNK
        if rem:
            pltpu.sync_copy(rows[0].at[pl.ds(0, rem)],
                            acc.at[pl.ds(zbase + nfull * _CHUNK, rem)])
        plsc.subcore_barrier()

        jbase = w * tpc

        def start_idx(j, k):
            pltpu.async_copy(edges.at[jbase + j], idx[k], isem[k])

        def wait_idx(j, k):
            pltpu.make_async_copy(edges.at[jbase + j], idx[k], isem[k]).wait()

        def start_gather(ki, kr):
            pltpu.async_copy(sup.at[idx[ki].at[0]], rows[kr], gsem[kr])

        def wait_gather(ki, kr):
            pltpu.make_async_copy(sup.at[idx[ki].at[0]],
                                  rows[kr], gsem[kr]).wait()

        def start_scatter(ki, kr):
            pltpu.async_copy(rows[kr], acc.at[idx[ki].at[1]],
                             ssem[kr], add=True)

        def wait_scatter(ki, kr):
            pltpu.make_async_copy(rows[kr], acc.at[idx[ki].at[1]],
                                  ssem[kr]).wait()

        def body(j, p):
            wait_scatter((p + 2) % 4, p % 2)
            start_idx(j + 2, (p + 2) % 4)
            wait_idx(j, p)
            start_gather(p, p % 2)
            wait_gather((p - 1) % 4, (p - 1) % 2)
            start_scatter((p - 1) % 4, (p - 1) % 2)

        for k in range(4):
            start_idx(k, k)
        wait_idx(0, 0)
        start_gather(0, 0)
        wait_idx(1, 1)
        start_gather(1, 1)
        wait_gather(0, 0)
        start_scatter(0, 0)
        body(2, 2)
        body(3, 3)

        def _quad(m, carry):
            j = 4 * m
            body(j, 0)
            body(j + 1, 1)
            body(j + 2, 2)
            body(j + 3, 3)
            return carry

        lax.fori_loop(1, tpc // 4, _quad, 0)
        wait_gather((tpc - 1) % 4, (tpc - 1) % 2)
        start_scatter((tpc - 1) % 4, (tpc - 1) % 2)
        wait_scatter((tpc - 2) % 4, (tpc - 2) % 2)
        wait_scatter((tpc - 1) % 4, (tpc - 1) % 2)
        wait_idx(tpc, tpc % 4)
        wait_idx(tpc + 1, (tpc + 1) % 4)

        plsc.subcore_barrier()
        out_base = c * _N + zbase
        last = _N - (_NS - 1) * _RPT

        @pl.when(s < _NS - 1)
        def _():
            pltpu.sync_copy(acc.at[pl.ds(zbase, _RPT)],
                            out.at[pl.ds(out_base, _RPT)])

        @pl.when(s == _NS - 1)
        def _():
            pltpu.sync_copy(acc.at[pl.ds(zbase, last)],
                            out.at[pl.ds(out_base, last)])

    return seg


def _tc_first(x, W, L, b):
    def body(x_ref, w_ref, l_ref, b_ref, sup_ref, init_ref):
        xb = x_ref[...]
        sup_ref[...] = jnp.dot(xb, w_ref[...],
                               preferred_element_type=jnp.float32)
        init_ref[...] = jnp.dot(xb, l_ref[...],
                                preferred_element_type=jnp.float32) + b_ref[...]

    nb = _N // _BR
    out = pl.pallas_call(
        body,
        grid=(nb,),
        in_specs=[
            pl.BlockSpec((_BR, _D), lambda i: (i, 0)),
            pl.BlockSpec((_D, _D), lambda i: (0, 0)),
            pl.BlockSpec((_D, _D), lambda i: (0, 0)),
            pl.BlockSpec((1, _D), lambda i: (0, 0)),
        ],
        out_specs=[
            pl.BlockSpec((_BR, _D), lambda i: (i, 0)),
            pl.BlockSpec((_BR, _D), lambda i: (i, 0)),
        ],
        out_shape=[
            jax.ShapeDtypeStruct((_N, _D), jnp.float32),
            jax.ShapeDtypeStruct((_N, _D), jnp.float32),
        ],
    )(x, W, L, b.reshape(1, _D))
    return out


def _tc_step(agg2, init_p, r, W, L, b, *, resid, want_z, want_mm):
    nb = _N // _BR

    def body(*refs):
        lo_ref, hi_ref, init_ref = refs[0], refs[1], refs[2]
        i = 3
        if resid:
            r_ref = refs[i]; i += 1
        if want_mm:
            w_ref, l_ref, b_ref = refs[i], refs[i + 1], refs[i + 2]
            i += 3
        outs = refs[i:]
        z = lo_ref[...] + hi_ref[...] + init_ref[...]
        if resid:
            z = (r_ref[...] + z) * 0.5
        o = 0
        if want_mm:
            outs[o][...] = jnp.dot(z, w_ref[...],
                                   preferred_element_type=jnp.float32)
            outs[o + 1][...] = jnp.dot(z, l_ref[...],
                                       preferred_element_type=jnp.float32) + b_ref[...]
            o += 2
        if want_z:
            outs[o][...] = z

    in_specs = [
        pl.BlockSpec((_BR, _D), lambda i: (i, 0)),
        pl.BlockSpec((_BR, _D), lambda i: (nb + i, 0)),
        pl.BlockSpec((_BR, _D), lambda i: (i, 0)),
    ]
    args = [agg2, agg2, init_p]
    if resid:
        in_specs.append(pl.BlockSpec((_BR, _D), lambda i: (i, 0)))
        args.append(r)
    if want_mm:
        in_specs += [
            pl.BlockSpec((_D, _D), lambda i: (0, 0)),
            pl.BlockSpec((_D, _D), lambda i: (0, 0)),
            pl.BlockSpec((1, _D), lambda i: (0, 0)),
        ]
        args += [W, L, b.reshape(1, _D)]
    n_out = (2 if want_mm else 0) + (1 if want_z else 0)
    out = pl.pallas_call(
        body,
        grid=(nb,),
        in_specs=in_specs,
        out_specs=[pl.BlockSpec((_BR, _D), lambda i: (i, 0))] * n_out,
        out_shape=[jax.ShapeDtypeStruct((_N, _D), jnp.float32)] * n_out,
    )(*args)
    return out


def kernel(x, edge_index, W1, L1, b1, Wb, Lb, bb, W2, L2, b2):
    src = edge_index[0].astype(jnp.int32)
    dst = edge_index[1].astype(jnp.int32)
    e = src.shape[0]
    nw = _NC * _NS
    tpc = 8 * (-(-e // (nw * _CHUNK * 8)))
    pad = nw * tpc * _CHUNK - e
    srcp = jnp.concatenate(
        [src, jnp.zeros((pad,), jnp.int32)]).reshape(nw * tpc, _CHUNK)
    dstp = jnp.concatenate(
        [dst, jnp.full((pad,), _N, jnp.int32)]).reshape(nw * tpc, _CHUNK)
    edges = jnp.concatenate(
        [jnp.stack([srcp, dstp], axis=1),
         jnp.zeros((8, 2, _CHUNK), jnp.int32)], axis=0)
    seg = _make_sc_seg(tpc)

    def sc(sup):
        return seg(sup, edges)

    sup, init = _tc_first(x, W1, L1, b1)
    agg = sc(sup)
    sup, init, z1 = _tc_step(agg, init, None, Wb[0], Lb[0], bb[0],
                             resid=False, want_z=True, want_mm=True)
    agg = sc(sup)
    sup, init = _tc_step(agg, init, None, Wb[1], Lb[1], bb[1],
                         resid=False, want_z=False, want_mm=True)
    agg = sc(sup)
    sup, init, z3 = _tc_step(agg, init, z1, Wb[2], Lb[2], bb[2],
                             resid=True, want_z=True, want_mm=True)
    agg = sc(sup)
    sup, init = _tc_step(agg, init, None, Wb[3], Lb[3], bb[3],
                         resid=False, want_z=False, want_mm=True)
    agg = sc(sup)
    sup, init, z5 = _tc_step(agg, init, z3, Wb[4], Lb[4], bb[4],
                             resid=True, want_z=True, want_mm=True)
    agg = sc(sup)
    sup, init = _tc_step(agg, init, None, Wb[5], Lb[5], bb[5],
                         resid=False, want_z=False, want_mm=True)
    agg = sc(sup)
    sup, init, x_cat = _tc_step(agg, init, z5, W2, L2, b2,
                                resid=True, want_z=True, want_mm=True)
    agg = sc(sup)
    (x_out,) = _tc_step(agg, init, None, None, None, None,
                        resid=False, want_z=True, want_mm=False)
    return (x_out, x_cat)


# speedup vs baseline: 1.0382x; 1.0382x over previous
import functools

import jax
import jax.numpy as jnp
from jax import lax
from jax.experimental import pallas as pl
from jax.experimental.pallas import tpu as pltpu
from jax.experimental.pallas import tpu_sc as plsc

_N = 10000
_D = 128
_NC = 2
_NS = 16
_CHUNK = 128
_RPT = 8 * (-(-_N // (_NS * 8)))
_N_ACC = _NS * _RPT
_BR = 1000


@functools.cache
def _make_sc_seg(tpc):
    mesh = plsc.VectorSubcoreMesh(core_axis_name="c", subcore_axis_name="s")

    @functools.partial(
        pl.kernel,
        out_type=jax.ShapeDtypeStruct((2 * _N, _D), jnp.float32),
        mesh=mesh,
        scratch_types=[
            [pltpu.VMEM((2, _CHUNK), jnp.int32)] * 4,
            [pltpu.VMEM((_CHUNK, _D), jnp.float32)] * 2,
            pltpu.VMEM_SHARED((_N_ACC, _D), jnp.float32),
            [pltpu.SemaphoreType.DMA] * 4,
            [pltpu.SemaphoreType.DMA] * 2,
            [pltpu.SemaphoreType.DMA] * 2,
        ],
    )
    def seg(sup, edges, out, idx, rows, acc, isem, gsem, ssem):
        c = lax.axis_index("c")
        s = lax.axis_index("s")
        w = c * _NS + s

        zero16 = jnp.zeros((16,), jnp.float32)

        def _zrow(r, carry):
            for k in range(_D // 16):
                rows[0][r, pl.ds(16 * k, 16)] = zero16
            return carry

        lax.fori_loop(0, _CHUNK, _zrow, 0)
        zbase = s * _RPT
        nfull = _RPT // _CHUNK
        for m in range(nfull):
            pltpu.sync_copy(rows[0], acc.at[pl.ds(zbase + m * _CHUNK, _CHUNK)])
        rem = _RPT % _CHUNK
        if rem:
            pltpu.sync_copy(rows[0].at[pl.ds(0, rem)],
                            acc.at[pl.ds(zbase + nfull * _CHUNK, rem)])
        plsc.subcore_barrier()

        jbase = w * tpc

        def start_idx(j, k):
            pltpu.async_copy(edges.at[jbase + j], idx[k], isem[k])

        def wait_idx(j, k):
            pltpu.make_async_copy(edges.at[jbase + j], idx[k], isem[k]).wait()

        def start_gather(ki, kr):
            pltpu.async_copy(sup.at[idx[ki].at[0]], rows[kr], gsem[kr])

        def wait_gather(ki, kr):
            pltpu.make_async_copy(sup.at[idx[ki].at[0]],
                                  rows[kr], gsem[kr]).wait()

        def start_scatter(ki, kr):
            pltpu.async_copy(rows[kr], acc.at[idx[ki].at[1]],
                             ssem[kr], add=True)

        def wait_scatter(ki, kr):
            pltpu.make_async_copy(rows[kr], acc.at[idx[ki].at[1]],
                                  ssem[kr]).wait()

        def body(j, p):
            wait_scatter((p + 2) % 4, p % 2)
            start_idx(j + 2, (p + 2) % 4)
            wait_idx(j, p)
            start_gather(p, p % 2)
            wait_gather((p - 1) % 4, (p - 1) % 2)
            start_scatter((p - 1) % 4, (p - 1) % 2)

        for k in range(4):
            start_idx(k, k)
        wait_idx(0, 0)
        start_gather(0, 0)
        wait_idx(1, 1)
        start_gather(1, 1)
        wait_gather(0, 0)
        start_scatter(0, 0)
        body(2, 2)
        body(3, 3)

        def _quad(m, carry):
            j = 4 * m
            body(j, 0)
            body(j + 1, 1)
            body(j + 2, 2)
            body(j + 3, 3)
            return carry

        lax.fori_loop(1, tpc // 4, _quad, 0)
        wait_gather((tpc - 1) % 4, (tpc - 1) % 2)
        start_scatter((tpc - 1) % 4, (tpc - 1) % 2)
        wait_scatter((tpc - 2) % 4, (tpc - 2) % 2)
        wait_scatter((tpc - 1) % 4, (tpc - 1) % 2)
        wait_idx(tpc, tpc % 4)
        wait_idx(tpc + 1, (tpc + 1) % 4)

        plsc.subcore_barrier()
        out_base = c * _N + zbase
        last = _N - (_NS - 1) * _RPT

        @pl.when(s < _NS - 1)
        def _():
            pltpu.sync_copy(acc.at[pl.ds(zbase, _RPT)],
                            out.at[pl.ds(out_base, _RPT)])

        @pl.when(s == _NS - 1)
        def _():
            pltpu.sync_copy(acc.at[pl.ds(zbase, last)],
                            out.at[pl.ds(out_base, last)])

    return seg


def _tc_first(x, W, L, b):
    def body(x_ref, w_ref, l_ref, b_ref, sup_ref, init_ref):
        xb = x_ref[...]
        sup_ref[...] = jnp.dot(xb, w_ref[...],
                               preferred_element_type=jnp.float32)
        init_ref[...] = jnp.dot(xb, l_ref[...],
                                preferred_element_type=jnp.float32) + b_ref[...]

    nb = _N // _BR
    out = pl.pallas_call(
        body,
        grid=(nb,),
        in_specs=[
            pl.BlockSpec((_BR, _D), lambda i: (i, 0)),
            pl.BlockSpec((_D, _D), lambda i: (0, 0)),
            pl.BlockSpec((_D, _D), lambda i: (0, 0)),
            pl.BlockSpec((1, _D), lambda i: (0, 0)),
        ],
        out_specs=[
            pl.BlockSpec((_BR, _D), lambda i: (i, 0)),
            pl.BlockSpec((_BR, _D), lambda i: (i, 0)),
        ],
        out_shape=[
            jax.ShapeDtypeStruct((_N, _D), jnp.float32),
            jax.ShapeDtypeStruct((_N, _D), jnp.float32),
        ],
    )(x, W, L, b.reshape(1, _D))
    return out


def _tc_step(agg2, init_p, r, W, L, b, *, resid, want_z, want_mm):
    nb = _N // _BR

    def body(*refs):
        lo_ref, hi_ref, init_ref = refs[0], refs[1], refs[2]
        i = 3
        if resid:
            r_ref = refs[i]; i += 1
        if want_mm:
            w_ref, l_ref, b_ref = refs[i], refs[i + 1], refs[i + 2]
            i += 3
        outs = refs[i:]
        z = lo_ref[...] + hi_ref[...] + init_ref[...]
        if resid:
            z = (r_ref[...] + z) * 0.5
        o = 0
        if want_mm:
            outs[o][...] = jnp.dot(z, w_ref[...],
                                   preferred_element_type=jnp.float32)
            outs[o + 1][...] = jnp.dot(z, l_ref[...],
                                       preferred_element_type=jnp.float32) + b_ref[...]
            o += 2
        if want_z:
            outs[o][...] = z

    in_specs = [
        pl.BlockSpec((_BR, _D), lambda i: (i, 0)),
        pl.BlockSpec((_BR, _D), lambda i: (nb + i, 0)),
        pl.BlockSpec((_BR, _D), lambda i: (i, 0)),
    ]
    args = [agg2, agg2, init_p]
    if resid:
        in_specs.append(pl.BlockSpec((_BR, _D), lambda i: (i, 0)))
        args.append(r)
    if want_mm:
        in_specs += [
            pl.BlockSpec((_D, _D), lambda i: (0, 0)),
            pl.BlockSpec((_D, _D), lambda i: (0, 0)),
            pl.BlockSpec((1, _D), lambda i: (0, 0)),
        ]
        args += [W, L, b.reshape(1, _D)]
    n_out = (2 if want_mm else 0) + (1 if want_z else 0)
    out = pl.pallas_call(
        body,
        grid=(nb,),
        in_specs=in_specs,
        out_specs=[pl.BlockSpec((_BR, _D), lambda i: (i, 0))] * n_out,
        out_shape=[jax.ShapeDtypeStruct((_N, _D), jnp.float32)] * n_out,
    )(*args)
    return out


def kernel(x, edge_index, W1, L1, b1, Wb, Lb, bb, W2, L2, b2):
    src = edge_index[0].astype(jnp.int32)
    dst = edge_index[1].astype(jnp.int32)
    e = src.shape[0]
    nw = _NC * _NS
    tpc = 8 * (-(-e // (nw * _CHUNK * 8)))
    pad = nw * tpc * _CHUNK - e
    srcp = jnp.concatenate(
        [src, jnp.zeros((pad,), jnp.int32)]).reshape(nw * tpc, _CHUNK)
    dstp = jnp.concatenate(
        [dst, _N + jnp.arange(pad, dtype=jnp.int32) % (_N_ACC - _N)]
    ).reshape(nw * tpc, _CHUNK)
    edges = jnp.concatenate(
        [jnp.stack([srcp, dstp], axis=1),
         jnp.zeros((8, 2, _CHUNK), jnp.int32)], axis=0)
    seg = _make_sc_seg(tpc)

    def sc(sup):
        return seg(sup, edges)

    sup, init = _tc_first(x, W1, L1, b1)
    agg = sc(sup)
    sup, init, z1 = _tc_step(agg, init, None, Wb[0], Lb[0], bb[0],
                             resid=False, want_z=True, want_mm=True)
    agg = sc(sup)
    sup, init = _tc_step(agg, init, None, Wb[1], Lb[1], bb[1],
                         resid=False, want_z=False, want_mm=True)
    agg = sc(sup)
    sup, init, z3 = _tc_step(agg, init, z1, Wb[2], Lb[2], bb[2],
                             resid=True, want_z=True, want_mm=True)
    agg = sc(sup)
    sup, init = _tc_step(agg, init, None, Wb[3], Lb[3], bb[3],
                         resid=False, want_z=False, want_mm=True)
    agg = sc(sup)
    sup, init, z5 = _tc_step(agg, init, z3, Wb[4], Lb[4], bb[4],
                             resid=True, want_z=True, want_mm=True)
    agg = sc(sup)
    sup, init = _tc_step(agg, init, None, Wb[5], Lb[5], bb[5],
                         resid=False, want_z=False, want_mm=True)
    agg = sc(sup)
    sup, init, x_cat = _tc_step(agg, init, z5, W2, L2, b2,
                                resid=True, want_z=True, want_mm=True)
    agg = sc(sup)
    (x_out,) = _tc_step(agg, init, None, None, None, None,
                        resid=False, want_z=True, want_mm=False)
    return (x_out, x_cat)


# speedup vs baseline: 1.0394x; 1.0012x over previous
import functools

import jax
import jax.numpy as jnp
from jax import lax
from jax.experimental import pallas as pl
from jax.experimental.pallas import tpu as pltpu
from jax.experimental.pallas import tpu_sc as plsc

_N = 10000
_D = 128
_NC = 2
_NS = 16
_CHUNK = 128
_RPT = 8 * (-(-_N // (_NS * 8)))
_N_ACC = _NS * _RPT
_BR = 1000


@functools.cache
def _make_sc_seg(tpc):
    mesh = plsc.VectorSubcoreMesh(core_axis_name="c", subcore_axis_name="s")

    @functools.partial(
        pl.kernel,
        out_type=jax.ShapeDtypeStruct((2 * _N, _D), jnp.float32),
        mesh=mesh,
        scratch_types=[
            [pltpu.VMEM((2, _CHUNK), jnp.int32)] * 4,
            [pltpu.VMEM((_CHUNK, _D), jnp.float32)] * 2,
            pltpu.VMEM_SHARED((_N_ACC, _D), jnp.float32),
            [pltpu.SemaphoreType.DMA] * 4,
            [pltpu.SemaphoreType.DMA] * 2,
            [pltpu.SemaphoreType.DMA] * 2,
            [pltpu.SemaphoreType.DMA] * 2,
        ],
    )
    def seg(sup, edges, out, idx, rows, acc, isem, gsem, gsem2, ssem):
        c = lax.axis_index("c")
        s = lax.axis_index("s")
        w = c * _NS + s

        zero16 = jnp.zeros((16,), jnp.float32)

        def _zrow(r, carry):
            for k in range(_D // 16):
                rows[0][r, pl.ds(16 * k, 16)] = zero16
            return carry

        lax.fori_loop(0, _CHUNK, _zrow, 0)
        zbase = s * _RPT
        nfull = _RPT // _CHUNK
        for m in range(nfull):
            pltpu.sync_copy(rows[0], acc.at[pl.ds(zbase + m * _CHUNK, _CHUNK)])
        rem = _RPT % _CHUNK
        if rem:
            pltpu.sync_copy(rows[0].at[pl.ds(0, rem)],
                            acc.at[pl.ds(zbase + nfull * _CHUNK, rem)])
        plsc.subcore_barrier()

        jbase = w * tpc

        def start_idx(j, k):
            pltpu.async_copy(edges.at[jbase + j], idx[k], isem[k])

        def wait_idx(j, k):
            pltpu.make_async_copy(edges.at[jbase + j], idx[k], isem[k]).wait()

        h = _CHUNK // 2

        def start_gather(ki, kr):
            pltpu.async_copy(sup.at[idx[ki].at[0, pl.ds(0, h)]],
                             rows[kr].at[pl.ds(0, h)], gsem[kr])
            pltpu.async_copy(sup.at[idx[ki].at[0, pl.ds(h, h)]],
                             rows[kr].at[pl.ds(h, h)], gsem2[kr])

        def wait_gather(ki, kr):
            pltpu.make_async_copy(sup.at[idx[ki].at[0, pl.ds(0, h)]],
                                  rows[kr].at[pl.ds(0, h)], gsem[kr]).wait()
            pltpu.make_async_copy(sup.at[idx[ki].at[0, pl.ds(h, h)]],
                                  rows[kr].at[pl.ds(h, h)], gsem2[kr]).wait()

        def start_scatter(ki, kr):
            pltpu.async_copy(rows[kr], acc.at[idx[ki].at[1]],
                             ssem[kr], add=True)

        def wait_scatter(ki, kr):
            pltpu.make_async_copy(rows[kr], acc.at[idx[ki].at[1]],
                                  ssem[kr]).wait()

        def body(j, p):
            wait_scatter((p + 2) % 4, p % 2)
            start_idx(j + 2, (p + 2) % 4)
            wait_idx(j, p)
            start_gather(p, p % 2)
            wait_gather((p - 1) % 4, (p - 1) % 2)
            start_scatter((p - 1) % 4, (p - 1) % 2)

        for k in range(4):
            start_idx(k, k)
        wait_idx(0, 0)
        start_gather(0, 0)
        wait_idx(1, 1)
        start_gather(1, 1)
        wait_gather(0, 0)
        start_scatter(0, 0)
        body(2, 2)
        body(3, 3)

        def _quad(m, carry):
            j = 4 * m
            body(j, 0)
            body(j + 1, 1)
            body(j + 2, 2)
            body(j + 3, 3)
            return carry

        lax.fori_loop(1, tpc // 4, _quad, 0)
        wait_gather((tpc - 1) % 4, (tpc - 1) % 2)
        start_scatter((tpc - 1) % 4, (tpc - 1) % 2)
        wait_scatter((tpc - 2) % 4, (tpc - 2) % 2)
        wait_scatter((tpc - 1) % 4, (tpc - 1) % 2)
        wait_idx(tpc, tpc % 4)
        wait_idx(tpc + 1, (tpc + 1) % 4)

        plsc.subcore_barrier()
        out_base = c * _N + zbase
        last = _N - (_NS - 1) * _RPT

        @pl.when(s < _NS - 1)
        def _():
            pltpu.sync_copy(acc.at[pl.ds(zbase, _RPT)],
                            out.at[pl.ds(out_base, _RPT)])

        @pl.when(s == _NS - 1)
        def _():
            pltpu.sync_copy(acc.at[pl.ds(zbase, last)],
                            out.at[pl.ds(out_base, last)])

    return seg


def _tc_first(x, W, L, b):
    def body(x_ref, w_ref, l_ref, b_ref, sup_ref, init_ref):
        xb = x_ref[...]
        sup_ref[...] = jnp.dot(xb, w_ref[...],
                               preferred_element_type=jnp.float32)
        init_ref[...] = jnp.dot(xb, l_ref[...],
                                preferred_element_type=jnp.float32) + b_ref[...]

    nb = _N // _BR
    out = pl.pallas_call(
        body,
        grid=(nb,),
        in_specs=[
            pl.BlockSpec((_BR, _D), lambda i: (i, 0)),
            pl.BlockSpec((_D, _D), lambda i: (0, 0)),
            pl.BlockSpec((_D, _D), lambda i: (0, 0)),
            pl.BlockSpec((1, _D), lambda i: (0, 0)),
        ],
        out_specs=[
            pl.BlockSpec((_BR, _D), lambda i: (i, 0)),
            pl.BlockSpec((_BR, _D), lambda i: (i, 0)),
        ],
        out_shape=[
            jax.ShapeDtypeStruct((_N, _D), jnp.float32),
            jax.ShapeDtypeStruct((_N, _D), jnp.float32),
        ],
    )(x, W, L, b.reshape(1, _D))
    return out


def _tc_step(agg2, init_p, r, W, L, b, *, resid, want_z, want_mm):
    nb = _N // _BR

    def body(*refs):
        lo_ref, hi_ref, init_ref = refs[0], refs[1], refs[2]
        i = 3
        if resid:
            r_ref = refs[i]; i += 1
        if want_mm:
            w_ref, l_ref, b_ref = refs[i], refs[i + 1], refs[i + 2]
            i += 3
        outs = refs[i:]
        z = lo_ref[...] + hi_ref[...] + init_ref[...]
        if resid:
            z = (r_ref[...] + z) * 0.5
        o = 0
        if want_mm:
            outs[o][...] = jnp.dot(z, w_ref[...],
                                   preferred_element_type=jnp.float32)
            outs[o + 1][...] = jnp.dot(z, l_ref[...],
                                       preferred_element_type=jnp.float32) + b_ref[...]
            o += 2
        if want_z:
            outs[o][...] = z

    in_specs = [
        pl.BlockSpec((_BR, _D), lambda i: (i, 0)),
        pl.BlockSpec((_BR, _D), lambda i: (nb + i, 0)),
        pl.BlockSpec((_BR, _D), lambda i: (i, 0)),
    ]
    args = [agg2, agg2, init_p]
    if resid:
        in_specs.append(pl.BlockSpec((_BR, _D), lambda i: (i, 0)))
        args.append(r)
    if want_mm:
        in_specs += [
            pl.BlockSpec((_D, _D), lambda i: (0, 0)),
            pl.BlockSpec((_D, _D), lambda i: (0, 0)),
            pl.BlockSpec((1, _D), lambda i: (0, 0)),
        ]
        args += [W, L, b.reshape(1, _D)]
    n_out = (2 if want_mm else 0) + (1 if want_z else 0)
    out = pl.pallas_call(
        body,
        grid=(nb,),
        in_specs=in_specs,
        out_specs=[pl.BlockSpec((_BR, _D), lambda i: (i, 0))] * n_out,
        out_shape=[jax.ShapeDtypeStruct((_N, _D), jnp.float32)] * n_out,
    )(*args)
    return out


def kernel(x, edge_index, W1, L1, b1, Wb, Lb, bb, W2, L2, b2):
    src = edge_index[0].astype(jnp.int32)
    dst = edge_index[1].astype(jnp.int32)
    e = src.shape[0]
    nw = _NC * _NS
    tpc = 8 * (-(-e // (nw * _CHUNK * 8)))
    pad = nw * tpc * _CHUNK - e
    srcp = jnp.concatenate(
        [src, jnp.zeros((pad,), jnp.int32)]).reshape(nw * tpc, _CHUNK)
    dstp = jnp.concatenate(
        [dst, _N + jnp.arange(pad, dtype=jnp.int32) % (_N_ACC - _N)]
    ).reshape(nw * tpc, _CHUNK)
    edges = jnp.concatenate(
        [jnp.stack([srcp, dstp], axis=1),
         jnp.zeros((8, 2, _CHUNK), jnp.int32)], axis=0)
    seg = _make_sc_seg(tpc)

    def sc(sup):
        return seg(sup, edges)

    sup, init = _tc_first(x, W1, L1, b1)
    agg = sc(sup)
    sup, init, z1 = _tc_step(agg, init, None, Wb[0], Lb[0], bb[0],
                             resid=False, want_z=True, want_mm=True)
    agg = sc(sup)
    sup, init = _tc_step(agg, init, None, Wb[1], Lb[1], bb[1],
                         resid=False, want_z=False, want_mm=True)
    agg = sc(sup)
    sup, init, z3 = _tc_step(agg, init, z1, Wb[2], Lb[2], bb[2],
                             resid=True, want_z=True, want_mm=True)
    agg = sc(sup)
    sup, init = _tc_step(agg, init, None, Wb[3], Lb[3], bb[3],
                         resid=False, want_z=False, want_mm=True)
    agg = sc(sup)
    sup, init, z5 = _tc_step(agg, init, z3, Wb[4], Lb[4], bb[4],
                             resid=True, want_z=True, want_mm=True)
    agg = sc(sup)
    sup, init = _tc_step(agg, init, None, Wb[5], Lb[5], bb[5],
                         resid=False, want_z=False, want_mm=True)
    agg = sc(sup)
    sup, init, x_cat = _tc_step(agg, init, z5, W2, L2, b2,
                                resid=True, want_z=True, want_mm=True)
    agg = sc(sup)
    (x_out,) = _tc_step(agg, init, None, None, None, None,
                        resid=False, want_z=True, want_mm=False)
    return (x_out, x_cat)
